# baseline (speedup 1.0000x reference)
import functools

import jax
import jax.numpy as jnp
from jax.experimental import pallas as pl
from jax.experimental.pallas import tpu as pltpu

EPS = 1e-6


def _moe_body(x_ref, rmsw_ref, rw_ref, w1_ref, w2_ref, o_ref, *, n_e):
    e = pl.program_id(1)
    f = pl.program_id(2)
    xb = x_ref[...]
    xn = xb * jax.lax.rsqrt(jnp.mean(xb * xb, axis=-1, keepdims=True) + EPS)
    xn = xn * rmsw_ref[...]
    scores = jnp.dot(xn, rw_ref[...].T, preferred_element_type=jnp.float32)
    rb = scores.shape[0]
    iota_e = jax.lax.broadcasted_iota(jnp.int32, (rb, n_e), 1)
    m0 = jnp.max(scores, axis=1, keepdims=True)
    i0 = jnp.min(jnp.where(scores == m0, iota_e, n_e), axis=1, keepdims=True)
    masked = jnp.where(iota_e == i0, -1e30, scores)
    m1 = jnp.max(masked, axis=1, keepdims=True)
    i1 = jnp.min(jnp.where(masked == m1, iota_e, n_e), axis=1, keepdims=True)
    p0 = 1.0 / (1.0 + jnp.exp(m1 - m0))
    p1 = 1.0 - p0
    coef = p0 * (i0 == e).astype(jnp.float32) + p1 * (i1 == e).astype(jnp.float32)
    h = jnp.dot(xn, w1_ref[0].T, preferred_element_type=jnp.float32)
    h = h * (1.0 / (1.0 + jnp.exp(-h)))
    yb = jnp.dot(h, w2_ref[0].T, preferred_element_type=jnp.float32)
    contrib = yb * coef

    @pl.when((e == 0) & (f == 0))
    def _():
        o_ref[...] = xb + contrib

    @pl.when((e != 0) | (f != 0))
    def _():
        o_ref[...] += contrib


def kernel(x, rms_w, router_w, w1, w2):
    s, d = x.shape
    n_e, d_ff, _ = w1.shape
    rblk = 256
    fblk = 1024
    grid = (s // rblk, n_e, d_ff // fblk)
    out = pl.pallas_call(
        functools.partial(_moe_body, n_e=n_e),
        grid=grid,
        in_specs=[
            pl.BlockSpec((rblk, d), lambda r, e, f: (r, 0)),
            pl.BlockSpec((1, d), lambda r, e, f: (0, 0)),
            pl.BlockSpec((n_e, d), lambda r, e, f: (0, 0)),
            pl.BlockSpec((1, fblk, d), lambda r, e, f: (e, f, 0)),
            pl.BlockSpec((1, d, fblk), lambda r, e, f: (e, 0, f)),
        ],
        out_specs=pl.BlockSpec((rblk, d), lambda r, e, f: (r, 0)),
        out_shape=jax.ShapeDtypeStruct((s, d), jnp.float32),
        compiler_params=pltpu.CompilerParams(
            dimension_semantics=("parallel", "arbitrary", "arbitrary"),
        ),
    )(x, rms_w.reshape(1, d), router_w, w1, w2)
    return out


# speedup vs baseline: 1.8683x; 1.8683x over previous
import functools

import jax
import jax.numpy as jnp
from jax import lax
from jax.experimental import pallas as pl
from jax.experimental.pallas import tpu as pltpu
from jax.experimental.pallas import tpu_sc as plsc

EPS = 1e-6
RBLK = 128
NBLK = 40
BE_PAD = 64


def _cumsum_rows(a):
    n = a.shape[0]
    s = 1
    while s < n:
        shifted = jnp.concatenate([jnp.zeros((s, a.shape[1]), a.dtype), a[:-s, :]], axis=0)
        a = a + shifted
        s *= 2
    return a


def _route_body(x_ref, rmsw_ref, rw_ref, xn_ref, d0_ref, d1_ref, p0_ref, p1_ref,
                be_ref, *, n_e):
    xb = x_ref[...]
    xn = xb * jax.lax.rsqrt(jnp.mean(xb * xb, axis=-1, keepdims=True) + EPS)
    xn = xn * rmsw_ref[...]
    xn_ref[...] = xn
    scores = jnp.dot(xn, rw_ref[...].T, preferred_element_type=jnp.float32)
    s = scores.shape[0]
    iota_e = jax.lax.broadcasted_iota(jnp.int32, (s, n_e), 1)
    m0 = jnp.max(scores, axis=1, keepdims=True)
    i0 = jnp.min(jnp.where(scores == m0, iota_e, n_e), axis=1, keepdims=True)
    masked = jnp.where(iota_e == i0, -1e30, scores)
    m1 = jnp.max(masked, axis=1, keepdims=True)
    i1 = jnp.min(jnp.where(masked == m1, iota_e, n_e), axis=1, keepdims=True)
    p0 = 1.0 / (1.0 + jnp.exp(m1 - m0))
    p0_ref[...] = p0
    p1_ref[...] = 1.0 - p0

    onehot0 = (iota_e == i0).astype(jnp.int32)
    onehot1 = (iota_e == i1).astype(jnp.int32)
    c0 = _cumsum_rows(onehot0)
    c1 = _cumsum_rows(onehot1)
    counts0 = c0[s - 1:s, :]
    counts = counts0 + c1[s - 1:s, :]
    pc = ((counts + (RBLK - 1)) // RBLK) * RBLK
    eidx_r = jax.lax.broadcasted_iota(jnp.int32, (n_e, n_e), 0)
    eidx_c = jax.lax.broadcasted_iota(jnp.int32, (n_e, n_e), 1)
    tri = (eidx_r < eidx_c).astype(jnp.float32)
    poff_f = jnp.dot(pc.astype(jnp.float32), tri, preferred_element_type=jnp.float32)
    poff = poff_f.astype(jnp.int32)

    d0_ref[...] = jnp.sum(onehot0 * (poff + c0 - 1), axis=1, keepdims=True)
    d1_ref[...] = jnp.sum(onehot1 * (poff + counts0 + c1 - 1), axis=1, keepdims=True)

    ones8 = jnp.ones((n_e, 1), jnp.float32)
    outer = jnp.dot(ones8, poff_f, preferred_element_type=jnp.float32)
    eye = (eidx_r == eidx_c).astype(jnp.float32)
    poff_col = jnp.sum(outer * eye, axis=1, keepdims=True)
    bstart = (jax.lax.broadcasted_iota(jnp.int32, (1, BE_PAD), 1) * RBLK).astype(jnp.float32)
    cmp = (poff_col <= bstart).astype(jnp.int32)
    be_ref[...] = jnp.sum(cmp, axis=0, keepdims=True) - 1


def _group_mm_body(be_ref, xg_ref, w1_ref, w2_ref, yg_ref):
    xb = xg_ref[...]
    h = jnp.dot(xb, w1_ref[0].T, preferred_element_type=jnp.float32)
    h = h * (1.0 / (1.0 + jnp.exp(-h)))
    yg_ref[...] = jnp.dot(h, w2_ref[0].T, preferred_element_type=jnp.float32)


def _combine_body(x_ref, a_ref, b_ref, pa_ref, pb_ref, o_ref):
    o_ref[...] = x_ref[...] + pa_ref[...] * a_ref[...] + pb_ref[...] * b_ref[...]


def kernel(x, rms_w, router_w, w1, w2):
    s, d = x.shape
    n_e, d_ff, _ = w1.shape
    pt = NBLK * RBLK

    xn, d0, d1, p0, p1, be = pl.pallas_call(
        functools.partial(_route_body, n_e=n_e),
        in_specs=[
            pl.BlockSpec((s, d), lambda: (0, 0)),
            pl.BlockSpec((1, d), lambda: (0, 0)),
            pl.BlockSpec((n_e, d), lambda: (0, 0)),
        ],
        out_specs=[
            pl.BlockSpec((s, d), lambda: (0, 0)),
            pl.BlockSpec((s, 1), lambda: (0, 0)),
            pl.BlockSpec((s, 1), lambda: (0, 0)),
            pl.BlockSpec((s, 1), lambda: (0, 0)),
            pl.BlockSpec((s, 1), lambda: (0, 0)),
            pl.BlockSpec((1, BE_PAD), lambda: (0, 0)),
        ],
        out_shape=[
            jax.ShapeDtypeStruct((s, d), jnp.float32),
            jax.ShapeDtypeStruct((s, 1), jnp.int32),
            jax.ShapeDtypeStruct((s, 1), jnp.int32),
            jax.ShapeDtypeStruct((s, 1), jnp.float32),
            jax.ShapeDtypeStruct((s, 1), jnp.float32),
            jax.ShapeDtypeStruct((1, BE_PAD), jnp.int32),
        ],
    )(x, rms_w.reshape(1, d), router_w)

    d0f = d0.reshape(s)
    d1f = d1.reshape(s)

    info = plsc.get_sparse_core_info()
    nw = info.num_cores * info.num_subcores
    tpw = s // nw
    mesh = plsc.VectorSubcoreMesh(core_axis_name="c", subcore_axis_name="s")

    @functools.partial(
        pl.kernel, mesh=mesh,
        out_type=jax.ShapeDtypeStruct((pt, d), jnp.float32),
        scratch_types=[
            pltpu.VMEM((tpw,), jnp.int32),
            pltpu.VMEM((tpw, d), jnp.float32),
            pltpu.SemaphoreType.DMA,
        ],
    )
    def _scatter_k(xn_hbm, d0_hbm, d1_hbm, xg_hbm, idx_v, rows_v, sem):
        wid = lax.axis_index("s") * info.num_cores + lax.axis_index("c")
        base = wid * tpw
        pltpu.sync_copy(xn_hbm.at[pl.ds(base, tpw)], rows_v)
        pltpu.sync_copy(d0_hbm.at[pl.ds(base, tpw)], idx_v)
        pltpu.async_copy(rows_v, xg_hbm.at[idx_v], sem).wait()
        pltpu.sync_copy(d1_hbm.at[pl.ds(base, tpw)], idx_v)
        pltpu.async_copy(rows_v, xg_hbm.at[idx_v], sem).wait()

    xg = _scatter_k(xn, d0f, d1f)

    yg = pl.pallas_call(
        _group_mm_body,
        grid_spec=pltpu.PrefetchScalarGridSpec(
            num_scalar_prefetch=1,
            grid=(NBLK,),
            in_specs=[
                pl.BlockSpec((RBLK, d), lambda i, be: (i, 0)),
                pl.BlockSpec((1, d_ff, d), lambda i, be: (be[i], 0, 0)),
                pl.BlockSpec((1, d, d_ff), lambda i, be: (be[i], 0, 0)),
            ],
            out_specs=pl.BlockSpec((RBLK, d), lambda i, be: (i, 0)),
        ),
        out_shape=jax.ShapeDtypeStruct((pt, d), jnp.float32),
        compiler_params=pltpu.CompilerParams(
            dimension_semantics=("arbitrary",),
        ),
    )(be.reshape(BE_PAD), xg, w1, w2)

    @functools.partial(
        pl.kernel, mesh=mesh,
        out_type=jax.ShapeDtypeStruct((2 * s, d), jnp.float32),
        scratch_types=[
            pltpu.VMEM((tpw,), jnp.int32),
            pltpu.VMEM((tpw, d), jnp.float32),
            pltpu.SemaphoreType.DMA,
        ],
    )
    def _gather_k(yg_hbm, d0_hbm, d1_hbm, yp_hbm, idx_v, rows_v, sem):
        wid = lax.axis_index("s") * info.num_cores + lax.axis_index("c")
        base = wid * tpw
        pltpu.sync_copy(d0_hbm.at[pl.ds(base, tpw)], idx_v)
        pltpu.async_copy(yg_hbm.at[idx_v], rows_v, sem).wait()
        pltpu.sync_copy(rows_v, yp_hbm.at[pl.ds(base, tpw)])
        pltpu.sync_copy(d1_hbm.at[pl.ds(base, tpw)], idx_v)
        pltpu.async_copy(yg_hbm.at[idx_v], rows_v, sem).wait()
        pltpu.sync_copy(rows_v, yp_hbm.at[pl.ds(s + base, tpw)])

    yp = _gather_k(yg, d0f, d1f)

    eblk = 256
    out = pl.pallas_call(
        _combine_body,
        grid=(s // eblk,),
        in_specs=[
            pl.BlockSpec((eblk, d), lambda r: (r, 0)),
            pl.BlockSpec((eblk, d), lambda r: (r, 0)),
            pl.BlockSpec((eblk, d), lambda r: (r + s // eblk, 0)),
            pl.BlockSpec((eblk, 1), lambda r: (r, 0)),
            pl.BlockSpec((eblk, 1), lambda r: (r, 0)),
        ],
        out_specs=pl.BlockSpec((eblk, d), lambda r: (r, 0)),
        out_shape=jax.ShapeDtypeStruct((s, d), jnp.float32),
    )(x, yp, yp, p0, p1)
    return out


# speedup vs baseline: 2.5818x; 1.3819x over previous
import functools

import jax
import jax.numpy as jnp
from jax import lax
from jax.experimental import pallas as pl
from jax.experimental.pallas import tpu as pltpu
from jax.experimental.pallas import tpu_sc as plsc

EPS = 1e-6
RBLK = 256
NBLK = 24
BE_PAD = 64


def _cumsum_rows(a):
    n = a.shape[0]
    s = 1
    while s < n:
        shifted = jnp.concatenate([jnp.zeros((s, a.shape[1]), a.dtype), a[:-s, :]], axis=0)
        a = a + shifted
        s *= 2
    return a


def _route_body(x_ref, rmsw_ref, rw_ref, xn_ref, d0_ref, d1_ref, p0_ref, p1_ref,
                be_ref, *, n_e):
    xb = x_ref[...]
    xn = xb * jax.lax.rsqrt(jnp.mean(xb * xb, axis=-1, keepdims=True) + EPS)
    xn = xn * rmsw_ref[...]
    xn_ref[...] = xn
    scores = jnp.dot(xn, rw_ref[...].T, preferred_element_type=jnp.float32)
    s = scores.shape[0]
    iota_e = jax.lax.broadcasted_iota(jnp.int32, (s, n_e), 1)
    m0 = jnp.max(scores, axis=1, keepdims=True)
    i0 = jnp.min(jnp.where(scores == m0, iota_e, n_e), axis=1, keepdims=True)
    masked = jnp.where(iota_e == i0, -1e30, scores)
    m1 = jnp.max(masked, axis=1, keepdims=True)
    i1 = jnp.min(jnp.where(masked == m1, iota_e, n_e), axis=1, keepdims=True)
    p0 = 1.0 / (1.0 + jnp.exp(m1 - m0))
    p0_ref[...] = p0
    p1_ref[...] = 1.0 - p0

    onehot0 = (iota_e == i0).astype(jnp.int32)
    onehot1 = (iota_e == i1).astype(jnp.int32)
    c0 = _cumsum_rows(onehot0)
    c1 = _cumsum_rows(onehot1)
    counts0 = c0[s - 1:s, :]
    counts = counts0 + c1[s - 1:s, :]
    pc = ((counts + (RBLK - 1)) // RBLK) * RBLK
    eidx_r = jax.lax.broadcasted_iota(jnp.int32, (n_e, n_e), 0)
    eidx_c = jax.lax.broadcasted_iota(jnp.int32, (n_e, n_e), 1)
    tri = (eidx_r < eidx_c).astype(jnp.float32)
    poff_f = jnp.dot(pc.astype(jnp.float32), tri, preferred_element_type=jnp.float32)
    poff = poff_f.astype(jnp.int32)

    d0_ref[...] = jnp.sum(onehot0 * (poff + c0 - 1), axis=1, keepdims=True)
    d1_ref[...] = jnp.sum(onehot1 * (poff + counts0 + c1 - 1), axis=1, keepdims=True)

    ones8 = jnp.ones((n_e, 1), jnp.float32)
    outer = jnp.dot(ones8, poff_f, preferred_element_type=jnp.float32)
    eye = (eidx_r == eidx_c).astype(jnp.float32)
    poff_col = jnp.sum(outer * eye, axis=1, keepdims=True)
    bstart = (jax.lax.broadcasted_iota(jnp.int32, (1, BE_PAD), 1) * RBLK).astype(jnp.float32)
    cmp = (poff_col <= bstart).astype(jnp.int32)
    be_ref[...] = jnp.sum(cmp, axis=0, keepdims=True) - 1


def _group_mm_body(be_ref, xg_ref, w1_ref, w2_ref, yg_ref):
    xb = xg_ref[...]
    h = jnp.dot(xb, w1_ref[0].T, preferred_element_type=jnp.float32)
    h = h * (1.0 / (1.0 + jnp.exp(-h)))
    yg_ref[...] = jnp.dot(h, w2_ref[0].T, preferred_element_type=jnp.float32)


def _combine_body(x_ref, a_ref, b_ref, pa_ref, pb_ref, o_ref):
    o_ref[...] = x_ref[...] + pa_ref[...] * a_ref[...] + pb_ref[...] * b_ref[...]


def kernel(x, rms_w, router_w, w1, w2):
    s, d = x.shape
    n_e, d_ff, _ = w1.shape
    pt = NBLK * RBLK

    xn, d0, d1, p0, p1, be = pl.pallas_call(
        functools.partial(_route_body, n_e=n_e),
        in_specs=[
            pl.BlockSpec((s, d), lambda: (0, 0)),
            pl.BlockSpec((1, d), lambda: (0, 0)),
            pl.BlockSpec((n_e, d), lambda: (0, 0)),
        ],
        out_specs=[
            pl.BlockSpec((s, d), lambda: (0, 0)),
            pl.BlockSpec((s, 1), lambda: (0, 0)),
            pl.BlockSpec((s, 1), lambda: (0, 0)),
            pl.BlockSpec((s, 1), lambda: (0, 0)),
            pl.BlockSpec((s, 1), lambda: (0, 0)),
            pl.BlockSpec((1, BE_PAD), lambda: (0, 0)),
        ],
        out_shape=[
            jax.ShapeDtypeStruct((s, d), jnp.float32),
            jax.ShapeDtypeStruct((s, 1), jnp.int32),
            jax.ShapeDtypeStruct((s, 1), jnp.int32),
            jax.ShapeDtypeStruct((s, 1), jnp.float32),
            jax.ShapeDtypeStruct((s, 1), jnp.float32),
            jax.ShapeDtypeStruct((1, BE_PAD), jnp.int32),
        ],
    )(x, rms_w.reshape(1, d), router_w)

    d0f = d0.reshape(s)
    d1f = d1.reshape(s)

    info = plsc.get_sparse_core_info()
    nw = info.num_cores * info.num_subcores
    tpw = s // nw
    mesh = plsc.VectorSubcoreMesh(core_axis_name="c", subcore_axis_name="s")

    @functools.partial(
        pl.kernel, mesh=mesh,
        out_type=jax.ShapeDtypeStruct((pt, d), jnp.float32),
        scratch_types=[
            pltpu.VMEM((tpw,), jnp.int32),
            pltpu.VMEM((tpw, d), jnp.float32),
            pltpu.SemaphoreType.DMA,
        ],
    )
    def _scatter_k(xn_hbm, d0_hbm, d1_hbm, xg_hbm, idx_v, rows_v, sem):
        wid = lax.axis_index("s") * info.num_cores + lax.axis_index("c")
        base = wid * tpw
        pltpu.sync_copy(xn_hbm.at[pl.ds(base, tpw)], rows_v)
        pltpu.sync_copy(d0_hbm.at[pl.ds(base, tpw)], idx_v)
        pltpu.async_copy(rows_v, xg_hbm.at[idx_v], sem).wait()
        pltpu.sync_copy(d1_hbm.at[pl.ds(base, tpw)], idx_v)
        pltpu.async_copy(rows_v, xg_hbm.at[idx_v], sem).wait()

    xg = _scatter_k(xn, d0f, d1f)

    yg = pl.pallas_call(
        _group_mm_body,
        grid_spec=pltpu.PrefetchScalarGridSpec(
            num_scalar_prefetch=1,
            grid=(NBLK,),
            in_specs=[
                pl.BlockSpec((RBLK, d), lambda i, be: (i, 0)),
                pl.BlockSpec((1, d_ff, d), lambda i, be: (be[i], 0, 0)),
                pl.BlockSpec((1, d, d_ff), lambda i, be: (be[i], 0, 0)),
            ],
            out_specs=pl.BlockSpec((RBLK, d), lambda i, be: (i, 0)),
        ),
        out_shape=jax.ShapeDtypeStruct((pt, d), jnp.float32),
        compiler_params=pltpu.CompilerParams(
            dimension_semantics=("arbitrary",),
        ),
    )(be.reshape(BE_PAD), xg, w1, w2)

    @functools.partial(
        pl.kernel, mesh=mesh,
        out_type=jax.ShapeDtypeStruct((2 * s, d), jnp.float32),
        scratch_types=[
            pltpu.VMEM((tpw,), jnp.int32),
            pltpu.VMEM((tpw, d), jnp.float32),
            pltpu.SemaphoreType.DMA,
        ],
    )
    def _gather_k(yg_hbm, d0_hbm, d1_hbm, yp_hbm, idx_v, rows_v, sem):
        wid = lax.axis_index("s") * info.num_cores + lax.axis_index("c")
        base = wid * tpw
        pltpu.sync_copy(d0_hbm.at[pl.ds(base, tpw)], idx_v)
        pltpu.async_copy(yg_hbm.at[idx_v], rows_v, sem).wait()
        pltpu.sync_copy(rows_v, yp_hbm.at[pl.ds(base, tpw)])
        pltpu.sync_copy(d1_hbm.at[pl.ds(base, tpw)], idx_v)
        pltpu.async_copy(yg_hbm.at[idx_v], rows_v, sem).wait()
        pltpu.sync_copy(rows_v, yp_hbm.at[pl.ds(s + base, tpw)])

    yp = _gather_k(yg, d0f, d1f)

    eblk = 256
    out = pl.pallas_call(
        _combine_body,
        grid=(s // eblk,),
        in_specs=[
            pl.BlockSpec((eblk, d), lambda r: (r, 0)),
            pl.BlockSpec((eblk, d), lambda r: (r, 0)),
            pl.BlockSpec((eblk, d), lambda r: (r + s // eblk, 0)),
            pl.BlockSpec((eblk, 1), lambda r: (r, 0)),
            pl.BlockSpec((eblk, 1), lambda r: (r, 0)),
        ],
        out_specs=pl.BlockSpec((eblk, d), lambda r: (r, 0)),
        out_shape=jax.ShapeDtypeStruct((s, d), jnp.float32),
    )(x, yp, yp, p0, p1)
    return out
